# Initial kernel scaffold; baseline (speedup 1.0000x reference)
import jax
import jax.numpy as jnp
from jax.experimental import pallas as pl


def kernel(x, W):
    raise NotImplementedError("write your pallas kernel here")


# speedup vs baseline: 2.4286x; 2.4286x over previous
import functools

import jax
import jax.numpy as jnp
from jax import lax
from jax.experimental import pallas as pl
from jax.experimental.pallas import tpu as pltpu
from jax.experimental.pallas import tpu_sc as plsc

NUM_TOKENS = 16384
INPUT_DIM = 2048
NUM_EXPERTS = 64
TOPK = 2

NUM_WORKERS = 32
BLK = NUM_TOKENS // NUM_WORKERS
LANES = 16
GROUPS = BLK // LANES


def _tc_logits_body(x_ref, w_ref, out_ref):
    out_ref[0, :, :] = lax.dot_general(
        w_ref[...], x_ref[...],
        dimension_numbers=(((1,), (1,)), ((), ())),
        preferred_element_type=jnp.float32,
    )


def _tc_logits(x, W):
    return pl.pallas_call(
        _tc_logits_body,
        grid=(NUM_WORKERS,),
        in_specs=[
            pl.BlockSpec((BLK, INPUT_DIM), lambda i: (i, 0)),
            pl.BlockSpec((NUM_EXPERTS, INPUT_DIM), lambda i: (0, 0)),
        ],
        out_specs=pl.BlockSpec((1, NUM_EXPERTS, BLK), lambda i: (i, 0, 0)),
        out_shape=jax.ShapeDtypeStruct(
            (NUM_WORKERS, NUM_EXPERTS, BLK), jnp.float32),
    )(x, W)


def _sc_route_body(lt_hbm, fw_hbm, ix_hbm, lt_v, fw_v, ix_v, sem):
    del sem
    c = lax.axis_index("c")
    s = lax.axis_index("s")
    wid = s * 2 + c
    pltpu.sync_copy(lt_hbm.at[wid], lt_v)

    lane = lax.iota(jnp.int32, LANES)

    def group(g, carry):
        gbase = g * (LANES * NUM_EXPERTS)
        for j in range(LANES * NUM_EXPERTS // LANES):
            fw_v[pl.ds(gbase + j * LANES, LANES)] = jnp.zeros(
                (LANES,), jnp.float32)
        t0 = g * LANES
        m1 = lt_v[0, pl.ds(t0, LANES)]
        i1 = jnp.zeros((LANES,), jnp.int32)
        m2 = jnp.full((LANES,), -jnp.inf, jnp.float32)
        i2 = jnp.zeros((LANES,), jnp.int32)
        for e in range(1, NUM_EXPERTS):
            v = lt_v[e, pl.ds(t0, LANES)]
            ev = jnp.full((LANES,), e, jnp.int32)
            gt1 = v > m1
            gt2 = v > m2
            m2 = jnp.where(gt2, jnp.where(gt1, m1, v), m2)
            i2 = jnp.where(gt2, jnp.where(gt1, i1, ev), i2)
            m1 = jnp.where(gt1, v, m1)
            i1 = jnp.where(gt1, ev, i1)
        ed = jnp.exp(m2 - m1)
        denom = 1.0 + ed
        w1 = 1.0 / denom
        w2 = ed / denom
        tokf = gbase + lane * NUM_EXPERTS
        plsc.store_scatter(fw_v, [tokf + i1], w1)
        plsc.store_scatter(fw_v, [tokf + i2], w2)
        tki = g * (LANES * TOPK) + lane * TOPK
        plsc.store_scatter(ix_v, [tki], i1)
        plsc.store_scatter(ix_v, [tki + 1], i2)
        return carry

    lax.fori_loop(0, GROUPS, group, 0)

    fw_n = BLK * NUM_EXPERTS
    ix_n = BLK * TOPK
    pltpu.sync_copy(fw_v, fw_hbm.at[pl.ds(wid * fw_n, fw_n)])
    pltpu.sync_copy(ix_v, ix_hbm.at[pl.ds(wid * ix_n, ix_n)])


@functools.cache
def _sc_route():
    return pl.kernel(
        _sc_route_body,
        out_type=(
            jax.ShapeDtypeStruct((NUM_TOKENS * NUM_EXPERTS,), jnp.float32),
            jax.ShapeDtypeStruct((NUM_TOKENS * TOPK,), jnp.int32),
        ),
        mesh=plsc.VectorSubcoreMesh(
            core_axis_name="c", subcore_axis_name="s",
            num_cores=2, num_subcores=16),
        scratch_types=[
            pltpu.VMEM((NUM_EXPERTS, BLK), jnp.float32),
            pltpu.VMEM((BLK * NUM_EXPERTS,), jnp.float32),
            pltpu.VMEM((BLK * TOPK,), jnp.int32),
            pltpu.SemaphoreType.DMA,
        ],
        compiler_params=pltpu.CompilerParams(needs_layout_passes=False),
    )


def kernel(x, W):
    lt = _tc_logits(x, W)
    fw_flat, ix_flat = _sc_route()(lt)
    return (fw_flat.reshape(NUM_TOKENS, NUM_EXPERTS),
            ix_flat.reshape(NUM_TOKENS, TOPK))
